# baseline (speedup 1.0000x reference)
import functools

import jax
import jax.numpy as jnp
from jax import lax
from jax.experimental import pallas as pl
from jax.experimental.pallas import tpu as pltpu
from jax.experimental.pallas import tpu_sc as plsc

L = 16
NC, NS = 2, 16
NW = NC * NS
C = 16
NBUF = 8
DEPTH = 4
EPS = 1e-12
N_SLICES = 1
TC_BLK = 2048


def _make_sc_gather(steps, D):
    mesh = plsc.VectorSubcoreMesh(core_axis_name="c", subcore_axis_name="s",
                                  num_cores=NC, num_subcores=NS)

    def body(ids_hbm, w_hbm, out_hbm, idx_v, b0, b1, b2, b3, b4, b5, b6, b7,
             g0, g1, g2, g3, g4, g5, g6, g7, o0, o1, o2, o3, o4, o5, o6, o7):
        cid = lax.axis_index("c")
        sid = lax.axis_index("s")
        wid = sid * NC + cid
        pltpu.sync_copy(ids_hbm.at[wid], idx_v)

        bufs = (b0, b1, b2, b3, b4, b5, b6, b7)
        gsems = (g0, g1, g2, g3, g4, g5, g6, g7)
        osems = (o0, o1, o2, o3, o4, o5, o6, o7)

        def gather(c, k):
            pltpu.async_copy(w_hbm.at[idx_v.at[c]], bufs[k], gsems[k])

        def phase(c, k):
            pltpu.make_async_copy(w_hbm.at[idx_v.at[c]], bufs[k],
                                  gsems[k]).wait()
            pltpu.async_copy(bufs[k], out_hbm.at[wid, c], osems[k])

            @pl.when(c + DEPTH >= NBUF)
            def _():
                kd = (k + DEPTH) % NBUF
                pltpu.make_async_copy(bufs[kd],
                                      out_hbm.at[wid, c + DEPTH - NBUF],
                                      osems[kd]).wait()

            @pl.when(c + DEPTH < steps)
            def _():
                gather(c + DEPTH, (k + DEPTH) % NBUF)

        for d in range(DEPTH):
            gather(d, d)

        def dstep(h, carry):
            c0 = NBUF * h
            for k in range(NBUF):
                phase(c0 + k, k)
            return carry

        lax.fori_loop(0, steps // NBUF, dstep, 0)
        for d in range(NBUF - DEPTH, NBUF):
            c = steps - NBUF + d
            pltpu.make_async_copy(bufs[c % NBUF], out_hbm.at[wid, c],
                                  osems[c % NBUF]).wait()

    return pl.kernel(
        body,
        out_type=jax.ShapeDtypeStruct((NW, steps, C, D), jnp.float32),
        mesh=mesh,
        compiler_params=pltpu.CompilerParams(needs_layout_passes=False),
        scratch_types=(
            [pltpu.VMEM((steps, C), jnp.int32)]
            + [pltpu.VMEM((C, D), jnp.float32)] * NBUF
            + [pltpu.SemaphoreType.DMA] * (2 * NBUF)
        ),
    )


def _tc_ln_kernel(x_ref, g_ref, b_ref, o_ref):
    v = x_ref[...]
    mean = jnp.mean(v, axis=1, keepdims=True)
    var = jnp.mean(v * v, axis=1, keepdims=True) - mean * mean
    rinv = lax.rsqrt(var + EPS)
    o_ref[...] = (v - mean) * rinv * g_ref[...] + b_ref[...]


def _make_tc_ln(R, D):
    grid = (R // TC_BLK,)
    return pl.pallas_call(
        _tc_ln_kernel,
        grid=grid,
        in_specs=[
            pl.BlockSpec((TC_BLK, D), lambda i: (i, 0)),
            pl.BlockSpec((1, D), lambda i: (0, 0)),
            pl.BlockSpec((1, D), lambda i: (0, 0)),
        ],
        out_specs=pl.BlockSpec((TC_BLK, D), lambda i: (i, 0)),
        out_shape=jax.ShapeDtypeStruct((R, D), jnp.float32),
        compiler_params=pltpu.CompilerParams(
            dimension_semantics=("arbitrary",)),
    )


def kernel(input_ids, W, gamma, beta):
    orig_shape = input_ids.shape
    B = input_ids.size
    _, D = W.shape
    b_slice = B // N_SLICES
    steps = b_slice // (NW * C)
    sc_gather = _make_sc_gather(steps, D)
    tc_ln = _make_tc_ln(b_slice, D)
    g2 = gamma.reshape(1, D)
    b2 = beta.reshape(1, D)
    ids = input_ids.reshape(N_SLICES, NW, steps, C).astype(jnp.int32)
    raws = [sc_gather(ids[si], W) for si in range(N_SLICES)]
    outs = [tc_ln(raw.reshape(b_slice, D), g2, b2) for raw in raws]
    out = jnp.concatenate(outs, axis=0)
    return out.reshape(*orig_shape, D)


# speedup vs baseline: 1.0087x; 1.0087x over previous
import functools

import jax
import jax.numpy as jnp
from jax import lax
from jax.experimental import pallas as pl
from jax.experimental.pallas import tpu as pltpu
from jax.experimental.pallas import tpu_sc as plsc

L = 16
NC, NS = 2, 16
NW = NC * NS
C = 32
NBUF = 4
EPS = 1e-12
N_SLICES = 1
TC_BLK = 2048


def _make_sc_gather(steps, D):
    mesh = plsc.VectorSubcoreMesh(core_axis_name="c", subcore_axis_name="s",
                                  num_cores=NC, num_subcores=NS)

    def body(ids_hbm, w_hbm, out_hbm, idx_v, b0, b1, b2, b3,
             g0, g1, g2, g3, o0, o1, o2, o3):
        cid = lax.axis_index("c")
        sid = lax.axis_index("s")
        wid = sid * NC + cid
        pltpu.sync_copy(ids_hbm.at[wid], idx_v)

        bufs = (b0, b1, b2, b3)
        gsems = (g0, g1, g2, g3)
        osems = (o0, o1, o2, o3)

        def gather(c, k):
            pltpu.async_copy(w_hbm.at[idx_v.at[c]], bufs[k], gsems[k])

        def phase(c, k):
            pltpu.make_async_copy(w_hbm.at[idx_v.at[c]], bufs[k],
                                  gsems[k]).wait()
            pltpu.async_copy(bufs[k], out_hbm.at[wid, c], osems[k])

            @pl.when(c >= 1)
            def _():
                k3 = (k + 3) % NBUF
                pltpu.make_async_copy(bufs[k3], out_hbm.at[wid, c - 1],
                                      osems[k3]).wait()

            @pl.when(c + 3 < steps)
            def _():
                gather(c + 3, (k + 3) % NBUF)

        gather(0, 0)
        gather(1, 1)
        gather(2, 2)

        def dstep(h, carry):
            c0 = NBUF * h
            for k in range(NBUF):
                phase(c0 + k, k)
            return carry

        lax.fori_loop(0, steps // NBUF, dstep, 0)
        pltpu.make_async_copy(bufs[(steps - 1) % NBUF],
                              out_hbm.at[wid, steps - 1],
                              osems[(steps - 1) % NBUF]).wait()

    return pl.kernel(
        body,
        out_type=jax.ShapeDtypeStruct((NW, steps, C, D), jnp.float32),
        mesh=mesh,
        compiler_params=pltpu.CompilerParams(needs_layout_passes=False),
        scratch_types=(
            [pltpu.VMEM((steps, C), jnp.int32)]
            + [pltpu.VMEM((C, D), jnp.float32)] * NBUF
            + [pltpu.SemaphoreType.DMA] * (2 * NBUF)
        ),
    )


def _tc_ln_kernel(x_ref, g_ref, b_ref, o_ref):
    v = x_ref[...]
    mean = jnp.mean(v, axis=1, keepdims=True)
    var = jnp.mean(v * v, axis=1, keepdims=True) - mean * mean
    rinv = lax.rsqrt(var + EPS)
    o_ref[...] = (v - mean) * rinv * g_ref[...] + b_ref[...]


def _make_tc_ln(R, D):
    grid = (R // TC_BLK,)
    return pl.pallas_call(
        _tc_ln_kernel,
        grid=grid,
        in_specs=[
            pl.BlockSpec((TC_BLK, D), lambda i: (i, 0)),
            pl.BlockSpec((1, D), lambda i: (0, 0)),
            pl.BlockSpec((1, D), lambda i: (0, 0)),
        ],
        out_specs=pl.BlockSpec((TC_BLK, D), lambda i: (i, 0)),
        out_shape=jax.ShapeDtypeStruct((R, D), jnp.float32),
        compiler_params=pltpu.CompilerParams(
            dimension_semantics=("arbitrary",)),
    )


def kernel(input_ids, W, gamma, beta):
    orig_shape = input_ids.shape
    B = input_ids.size
    _, D = W.shape
    b_slice = B // N_SLICES
    steps = b_slice // (NW * C)
    sc_gather = _make_sc_gather(steps, D)
    tc_ln = _make_tc_ln(b_slice, D)
    g2 = gamma.reshape(1, D)
    b2 = beta.reshape(1, D)
    ids = input_ids.reshape(N_SLICES, NW, steps, C).astype(jnp.int32)
    raws = [sc_gather(ids[si], W) for si in range(N_SLICES)]
    outs = [tc_ln(raw.reshape(b_slice, D), g2, b2) for raw in raws]
    out = jnp.concatenate(outs, axis=0)
    return out.reshape(*orig_shape, D)
